# baseline (speedup 1.0000x reference)
import functools

import jax
import jax.numpy as jnp
from jax import lax
from jax.experimental import pallas as pl
from jax.experimental.pallas import tpu as pltpu
from jax.experimental.pallas import tpu_sc as plsc


def _pick_chunk(ew: int, limit: int) -> int:
    for ck in range(min(ew, limit), 15, -1):
        if ew % ck == 0 and ck % 16 == 0:
            return ck
    return ew


@functools.partial(jax.jit, static_argnames=("t", "n", "e"))
def _smoothness_sc(z, ei, w, *, t, n, e):
    info = plsc.get_sparse_core_info()
    nw = info.num_cores * info.num_subcores
    ew = e // nw
    ck = _pick_chunk(ew, 4000)
    nchunks = ew // ck
    ngroups = ck // 16
    unroll = 8 if ngroups % 8 == 0 else (5 if ngroups % 5 == 0 else 1)
    tp = t // 2
    zk = _pick_chunk(n, 4000)
    znch = n // zk

    mesh = plsc.VectorSubcoreMesh(core_axis_name="c", subcore_axis_name="s")

    @functools.partial(
        pl.kernel,
        mesh=mesh,
        compiler_params=pltpu.CompilerParams(needs_layout_passes=False),
        out_type=(
            jax.ShapeDtypeStruct((nw, 16), jnp.float32),
            jax.ShapeDtypeStruct((tp * n,), jnp.int32),
            jax.ShapeDtypeStruct((e,), jnp.int32),
        ),
        scratch_types=[
            pltpu.VMEM((n,), jnp.int32),
            pltpu.VMEM((n,), jnp.int32),
            pltpu.VMEM((ck,), jnp.int32),
            pltpu.VMEM((ck,), jnp.int32),
            pltpu.VMEM((ck,), jnp.float32),
            pltpu.VMEM((ck,), jnp.float32),
            pltpu.VMEM((zk,), jnp.float32),
            pltpu.VMEM((zk,), jnp.float32),
            pltpu.VMEM((zk,), jnp.int32),
            pltpu.VMEM((16,), jnp.float32),
            pltpu.SemaphoreType.DMA,
            pltpu.SemaphoreType.DMA,
        ],
    )
    def body(z_hbm, ei_hbm, w_hbm, out_hbm, zp_hbm, rcp_hbm, ztab0, ztab1,
             rcb0, rcb1, wb0, wb1, ze, zo, pkb, accv, sem0, sem1):
        cid = lax.axis_index("c")
        sid = lax.axis_index("s")
        wid = sid * info.num_cores + cid
        ebase = wid * ew
        sems = (sem0, sem1)
        rcbufs, wbufs = (rcb0, rcb1), (wb0, wb1)

        @pl.when(sid < tp)
        def _():
            def zpk(k, _):
                nb = k * zk
                o0 = pl.multiple_of(2 * sid * n + nb, 8)
                o1 = pl.multiple_of((2 * sid + 1) * n + nb, 8)
                pltpu.sync_copy(z_hbm.at[pl.ds(o0, zk)], ze)
                pltpu.sync_copy(z_hbm.at[pl.ds(o1, zk)], zo)

                @plsc.parallel_loop(0, zk // 16, unroll=4)
                def _(g):
                    a = ze[pl.ds(g * 16, 16)]
                    b = zo[pl.ds(g * 16, 16)]
                    words = plsc.bitcast(
                        plsc.pack(a, b, format=plsc.PackFormat.INTERLEAVED),
                        jnp.int32)
                    pkb[pl.ds(g * 16, 16)] = words

                pltpu.sync_copy(pkb, zp_hbm.at[pl.ds(pl.multiple_of(sid * n + nb, 8), zk)])
                return 0

            lax.fori_loop(0, znch, zpk, 0)

        def rcpk(k, _):
            base = ebase + k * ck
            pltpu.sync_copy(ei_hbm.at[pl.ds(base, ck)], rcb0)
            pltpu.sync_copy(ei_hbm.at[pl.ds(e + base, ck)], rcb1)

            @plsc.parallel_loop(0, ngroups, unroll=4)
            def _(g):
                r = rcb0[pl.ds(g * 16, 16)]
                c = rcb1[pl.ds(g * 16, 16)]
                pkb[pl.ds(g * 16, 16)] = r | lax.shift_left(c, 16)

            pltpu.sync_copy(pkb, rcp_hbm.at[pl.ds(base, ck)])
            return 0

        lax.fori_loop(0, nchunks, rcpk, 0)

        plsc.subcore_barrier()

        def fire(k, buf):
            base = ebase + k * ck
            sem = sems[buf]
            return (
                pltpu.async_copy(rcp_hbm.at[pl.ds(base, ck)], rcbufs[buf], sem),
                pltpu.async_copy(w_hbm.at[pl.ds(base, ck)], wbufs[buf], sem),
            )

        acc = jnp.zeros((16,), jnp.float32)

        def pass_body(q, acc):
            pltpu.sync_copy(zp_hbm.at[pl.ds(pl.multiple_of(2 * q * n, 8), n)], ztab0)
            pltpu.sync_copy(zp_hbm.at[pl.ds(pl.multiple_of((2 * q + 1) * n, 8), n)], ztab1)
            handles = fire(0, 0)
            for k in range(nchunks):
                cur = k % 2
                if k + 1 < nchunks:
                    next_handles = fire(k + 1, 1 - cur)
                for h in handles:
                    h.wait()
                if k + 1 < nchunks:
                    handles = next_handles

                @plsc.parallel_loop(0, ngroups, unroll=unroll, carry=acc)
                def group_loop(g, acc):
                    rcv = rcbufs[cur][pl.ds(g * 16, 16)]
                    wv = wbufs[cur][pl.ds(g * 16, 16)]
                    ri = rcv & 0xFFFF
                    ci = lax.shift_right_logical(rcv, 16)
                    s = jnp.zeros((16,), jnp.float32)
                    for ztab in (ztab0, ztab1):
                        aw = plsc.load_gather(ztab, [ri])
                        bw = plsc.load_gather(ztab, [ci])
                        a0, a1 = plsc.unpack(plsc.bitcast(aw, jnp.bfloat16),
                                             format=plsc.PackFormat.INTERLEAVED)
                        b0, b1 = plsc.unpack(plsc.bitcast(bw, jnp.bfloat16),
                                             format=plsc.PackFormat.INTERLEAVED)
                        d0 = a0 - b0
                        d1 = a1 - b1
                        s = s + (d0 * d0 + d1 * d1)
                    return acc + wv * s

                acc = group_loop
            return acc

        acc = lax.fori_loop(0, tp // 2, pass_body, acc)

        accv[...] = acc
        pltpu.sync_copy(accv, out_hbm.at[wid])

    return body(z, ei, w)


def kernel(delta_z, edge_index, edge_weight):
    t, n, _ = delta_z.shape
    e = edge_weight.shape[0]
    z = delta_z.reshape(t * n)
    ei = edge_index.astype(jnp.int32).reshape(2 * e)
    partials, _, _ = _smoothness_sc(z, ei, edge_weight, t=t, n=n, e=e)
    return partials.sum() / jnp.float32(t * e)
